# Initial kernel scaffold; baseline (speedup 1.0000x reference)
import jax
import jax.numpy as jnp
from jax.experimental import pallas as pl


def kernel(x, iInd, jInd):
    raise NotImplementedError("write your pallas kernel here")


# speedup vs baseline: 3.9568x; 3.9568x over previous
import math

import jax
import jax.numpy as jnp
from jax import lax
from jax.experimental import pallas as pl
from jax.experimental.pallas import tpu as pltpu
from jax.experimental.pallas import tpu_sc as plsc

C = 128
CQ = C // 4
N = 10000
E = 320000
NC, NS = 2, 16
CHUNK = 128
NCHUNK = math.ceil(2 * E / (NS * CHUNK))
EP = NS * CHUNK * NCHUNK
ROWS_PT = 632
N_PAD = NS * ROWS_PT

_WB_CHUNKS = ((0, 128), (128, 128), (256, 128), (384, 128), (512, 120))


def _sc_body(xt_hbm, src_hbm, dst_hbm, s_out, cnt_out,
             src_v, dst_v, rows_v, ones_v, zrows_v, zcnt_v,
             s_sh, cnt_sh, sem):
    core = lax.axis_index("c")
    sub = lax.axis_index("s")

    zero16 = jnp.zeros((16,), jnp.float32)
    one16 = jnp.ones((16,), jnp.float32)

    def zrow(r, _):
        def zcol(cc, _):
            zrows_v[r, pl.ds(cc * 16, 16)] = zero16
            return 0
        return lax.fori_loop(0, CQ // 16, zcol, 0)
    lax.fori_loop(0, CHUNK, zrow, 0)

    def zcrow(r, _):
        zcnt_v[r, :] = zero16
        ones_v[r, :] = one16
        return 0
    lax.fori_loop(0, CHUNK, zcrow, 0)

    r0 = sub * ROWS_PT
    pltpu.sync_copy(dst_hbm.at[sub], dst_v)

    for p in range(2):
        for off, sz in _WB_CHUNKS:
            pltpu.sync_copy(zrows_v.at[pl.ds(0, sz)],
                            s_sh.at[pl.ds(r0 + off, sz)])
        if p == 0:
            for off, sz in _WB_CHUNKS:
                pltpu.sync_copy(zcnt_v.at[pl.ds(0, sz)],
                                cnt_sh.at[pl.ds(r0 + off, sz)])
        plsc.subcore_barrier()

        fi = (core * 2 + p) * NS + sub
        pltpu.sync_copy(src_hbm.at[fi], src_v)

        if p == 0:
            def body(c, _):
                pltpu.async_copy(xt_hbm.at[src_v.at[c]], rows_v, sem).wait()
                pltpu.sync_copy(rows_v, s_sh.at[dst_v.at[c]], add=True)
                pltpu.sync_copy(ones_v, cnt_sh.at[dst_v.at[c]], add=True)
                return 0
        else:
            def body(c, _):
                pltpu.async_copy(xt_hbm.at[src_v.at[c]], rows_v, sem).wait()
                pltpu.sync_copy(rows_v, s_sh.at[dst_v.at[c]], add=True)
                return 0
        lax.fori_loop(0, NCHUNK, body, 0)

        plsc.subcore_barrier()

        qbase = (core * 2 + p) * N_PAD + r0
        for off, sz in _WB_CHUNKS:
            pltpu.sync_copy(s_sh.at[pl.ds(r0 + off, sz)],
                            rows_v.at[pl.ds(0, sz)])
            pltpu.sync_copy(rows_v.at[pl.ds(0, sz)],
                            s_out.at[pl.ds(qbase + off, sz)])
        if p == 0:
            cbase = core * N_PAD + r0
            for off, sz in _WB_CHUNKS:
                pltpu.sync_copy(cnt_sh.at[pl.ds(r0 + off, sz)],
                                zcnt_v.at[pl.ds(0, sz)])
                pltpu.sync_copy(zcnt_v.at[pl.ds(0, sz)],
                                cnt_out.at[pl.ds(cbase + off, sz)])
        plsc.subcore_barrier()


_sc_accumulate = pl.kernel(
    _sc_body,
    out_type=[
        jax.ShapeDtypeStruct((4 * N_PAD, CQ), jnp.float32),
        jax.ShapeDtypeStruct((NC * N_PAD, 16), jnp.float32),
    ],
    mesh=plsc.VectorSubcoreMesh(
        core_axis_name="c", subcore_axis_name="s",
        num_cores=NC, num_subcores=NS),
    scratch_types=[
        pltpu.VMEM((NCHUNK, CHUNK), jnp.int32),
        pltpu.VMEM((NCHUNK, CHUNK), jnp.int32),
        pltpu.VMEM((CHUNK, CQ), jnp.float32),
        pltpu.VMEM((CHUNK, 16), jnp.float32),
        pltpu.VMEM((CHUNK, CQ), jnp.float32),
        pltpu.VMEM((CHUNK, 16), jnp.float32),
        pltpu.VMEM_SHARED((N_PAD, CQ), jnp.float32),
        pltpu.VMEM_SHARED((N_PAD, 16), jnp.float32),
        pltpu.SemaphoreType.DMA,
    ],
    compiler_params=pltpu.CompilerParams(use_tc_tiling_on_sc=False),
)


def _combine_body(x0, x1, x2, x3, s0, s1, s2, s3, cnt_ref, o_ref):
    cnt = cnt_ref[:, 0:1]
    x = jnp.concatenate([x0[...], x1[...], x2[...], x3[...]], axis=1)
    s = jnp.concatenate([s0[...], s1[...], s2[...], s3[...]], axis=1)
    o_ref[...] = cnt * x - s


_NB = N_PAD // 128

_combine = pl.pallas_call(
    _combine_body,
    grid=(_NB,),
    in_specs=(
        [pl.BlockSpec((128, CQ), lambda i, q=q: (i + q * _NB, 0))
         for q in range(4)] * 2 +
        [pl.BlockSpec((128, 16), lambda i: (i, 0))]
    ),
    out_specs=pl.BlockSpec((128, C), lambda i: (i, 0)),
    out_shape=jax.ShapeDtypeStruct((N_PAD, C), jnp.float32),
)


@jax.jit
def kernel(x, iInd, jInd):
    xt = jnp.zeros((N_PAD, C), jnp.float32).at[:N].set(x[0].T)
    xtc = xt.reshape(N_PAD, 4, CQ).transpose(1, 0, 2).reshape(4 * N_PAD, CQ)

    pad = jnp.full((EP - 2 * E,), N, jnp.int32)
    src0 = jnp.concatenate([jInd, iInd, pad])
    src = (src0[None] + N_PAD * jnp.arange(4, dtype=jnp.int32)[:, None])
    src = src.reshape(NC, 2, NS, NCHUNK, CHUNK).reshape(4 * NS, NCHUNK, CHUNK)
    dst = jnp.concatenate([iInd, jInd, pad]).reshape(NS, NCHUNK, CHUNK)

    s_p, cnt_p = _sc_accumulate(xtc, src, dst)
    out_t = _combine(xtc, xtc, xtc, xtc, s_p, s_p, s_p, s_p, cnt_p)
    return out_t[:N].T[None]


# speedup vs baseline: 4.9356x; 1.2474x over previous
import math

import jax
import jax.numpy as jnp
from jax import lax
from jax.experimental import pallas as pl
from jax.experimental.pallas import tpu as pltpu
from jax.experimental.pallas import tpu_sc as plsc

C = 128
CQ = C // 4
SW = CQ + 8
N = 10000
E = 320000
NC, NS = 2, 16
CHUNK = 128
NCHUNK = 3 * math.ceil(2 * E / (NS * CHUNK * 3))
EP = NS * CHUNK * NCHUNK
ROWS_PT = 626
N_PAD = NS * ROWS_PT

_WB_CHUNKS = ((0, 128), (128, 128), (256, 128), (384, 128), (512, 114))


def _sc_body(xt_hbm, src_hbm, dst_hbm, s_out,
             src_v, dst_v, rows0_v, rows1_v, rows2_v, zrows_v,
             s_sh, g0, g1, g2, t0, t1, t2):
    core = lax.axis_index("c")
    sub = lax.axis_index("s")
    bufs = (rows0_v, rows1_v, rows2_v)
    gs = (g0, g1, g2)
    ts = (t0, t1, t2)

    zero16 = jnp.zeros((16,), jnp.float32)

    def zrow(r, _):
        zrows_v[r, pl.ds(0, 16)] = zero16
        zrows_v[r, pl.ds(16, 16)] = zero16
        zrows_v[r, pl.ds(24, 16)] = zero16
        return 0
    lax.fori_loop(0, CHUNK, zrow, 0)

    r0 = sub * ROWS_PT
    pltpu.sync_copy(dst_hbm.at[sub], dst_v)

    def gwait(buf, s):
        pltpu.make_async_copy(xt_hbm.at[pl.ds(0, CHUNK)], buf, s).wait()

    for p in range(2):
        for off, sz in _WB_CHUNKS:
            pltpu.sync_copy(zrows_v.at[pl.ds(0, sz)],
                            s_sh.at[pl.ds(r0 + off, sz)])
        plsc.subcore_barrier()

        fi = (core * 2 + p) * NS + sub
        pltpu.sync_copy(src_hbm.at[fi], src_v)

        for k in range(3):
            pltpu.async_copy(xt_hbm.at[src_v.at[k]], bufs[k], gs[k])

        def body3(h, _):
            c = h * 3
            for k in range(3):
                gwait(bufs[k], gs[k])
                pltpu.async_copy(bufs[k], s_sh.at[dst_v.at[c + k]], ts[k],
                                 add=True)
            for k in range(3):
                gwait(bufs[k], ts[k])
                nxt = jnp.minimum(c + k + 3, NCHUNK - 1)
                pltpu.async_copy(xt_hbm.at[src_v.at[nxt]], bufs[k], gs[k])
            return 0
        lax.fori_loop(0, NCHUNK // 3, body3, 0)
        for k in range(3):
            gwait(bufs[k], gs[k])

        plsc.subcore_barrier()

        qbase = (core * 2 + p) * N_PAD + r0
        for off, sz in _WB_CHUNKS:
            pltpu.sync_copy(s_sh.at[pl.ds(r0 + off, sz)],
                            rows0_v.at[pl.ds(0, sz)])
            pltpu.sync_copy(rows0_v.at[pl.ds(0, sz)],
                            s_out.at[pl.ds(qbase + off, sz)])
        plsc.subcore_barrier()


_sc_accumulate = pl.kernel(
    _sc_body,
    out_type=jax.ShapeDtypeStruct((4 * N_PAD, SW), jnp.float32),
    mesh=plsc.VectorSubcoreMesh(
        core_axis_name="c", subcore_axis_name="s",
        num_cores=NC, num_subcores=NS),
    scratch_types=[
        pltpu.VMEM((NCHUNK, CHUNK), jnp.int32),
        pltpu.VMEM((NCHUNK, CHUNK), jnp.int32),
        pltpu.VMEM((CHUNK, SW), jnp.float32),
        pltpu.VMEM((CHUNK, SW), jnp.float32),
        pltpu.VMEM((CHUNK, SW), jnp.float32),
        pltpu.VMEM((CHUNK, SW), jnp.float32),
        pltpu.VMEM_SHARED((N_PAD, SW), jnp.float32),
        pltpu.SemaphoreType.DMA,
        pltpu.SemaphoreType.DMA,
        pltpu.SemaphoreType.DMA,
        pltpu.SemaphoreType.DMA,
        pltpu.SemaphoreType.DMA,
        pltpu.SemaphoreType.DMA,
    ],
    compiler_params=pltpu.CompilerParams(use_tc_tiling_on_sc=False),
)

_NBLK = 128
_NB = -(-N_PAD // _NBLK)


def _combine_body(x_ref, s_ref, eye_ref, o_ref):
    ident = eye_ref[...]
    dn = (((0,), (0,)), ((), ()))
    hp = jax.lax.Precision.HIGHEST
    st = [lax.dot_general(s_ref[q], ident, dn, precision=hp)
          for q in range(4)]
    feats = jnp.concatenate([t[:CQ] for t in st], axis=0)
    cnt_t = st[0][CQ:CQ + 1]
    o_ref[...] = (cnt_t * x_ref[0] - feats)[None]


_combine = pl.pallas_call(
    _combine_body,
    grid=(_NB,),
    in_specs=[
        pl.BlockSpec((1, C, _NBLK), lambda i: (0, 0, i)),
        pl.BlockSpec((4, _NBLK, SW), lambda i: (0, i, 0)),
        pl.BlockSpec((_NBLK, _NBLK), lambda i: (0, 0)),
    ],
    out_specs=pl.BlockSpec((1, C, _NBLK), lambda i: (0, 0, i)),
    out_shape=jax.ShapeDtypeStruct((1, C, N), jnp.float32),
)


@jax.jit
def kernel(x, iInd, jInd):
    xt = jnp.zeros((N_PAD, C), jnp.float32).at[:N].set(x[0].T)
    xq = xt.reshape(N_PAD, 4, CQ).transpose(1, 0, 2)
    onecol = (jnp.arange(N_PAD) < N).astype(jnp.float32)[None, :, None]
    onecol = jnp.broadcast_to(onecol, (4, N_PAD, 1))
    zcol = jnp.zeros((4, N_PAD, SW - CQ - 1), jnp.float32)
    xtc = jnp.concatenate([xq, onecol, zcol], axis=2).reshape(4 * N_PAD, SW)

    pad = jnp.full((EP - 2 * E,), N, jnp.int32)
    src0 = jnp.concatenate([jInd, iInd, pad])
    src = (src0[None] + N_PAD * jnp.arange(4, dtype=jnp.int32)[:, None])
    src = src.reshape(4 * NS, NCHUNK, CHUNK)
    dst = jnp.concatenate([iInd, jInd, pad]).reshape(NS, NCHUNK, CHUNK)

    s_p = _sc_accumulate(xtc, src, dst)
    s_q = s_p.reshape(4, N_PAD, SW)
    eye = jnp.eye(_NBLK, dtype=jnp.float32)
    return _combine(x, s_q, eye)


# speedup vs baseline: 5.6466x; 1.1440x over previous
import math

import jax
import jax.numpy as jnp
from jax import lax
from jax.experimental import pallas as pl
from jax.experimental.pallas import tpu as pltpu
from jax.experimental.pallas import tpu_sc as plsc

C = 128
CQ = C // 4
N = 10000
E = 320000
NC, NS = 2, 16
CHUNK = 128
NCHUNK = 2 * math.ceil(2 * E / (NS * CHUNK * 2))
EP = NS * CHUNK * NCHUNK
ROWS_PT = 626
N_PAD = NS * ROWS_PT

_WB_CHUNKS = ((0, 128), (128, 128), (256, 128), (384, 128), (512, 114))


def _sc_body(xt_hbm, src_hbm, dst_hbm, s_out, cnt_out,
             src_v, dst_v, rows0_v, rows1_v, ones_v, zrows_v, zcnt_v,
             s_sh, cnt_sh, g0, g1):
    core = lax.axis_index("c")
    sub = lax.axis_index("s")

    zero16 = jnp.zeros((16,), jnp.float32)
    one16 = jnp.ones((16,), jnp.float32)

    def zrow(r, _):
        zrows_v[r, pl.ds(0, 16)] = zero16
        zrows_v[r, pl.ds(16, 16)] = zero16
        return 0
    lax.fori_loop(0, CHUNK, zrow, 0)

    def zcrow(r, _):
        zcnt_v[r, :] = zero16
        ones_v[r, :] = one16
        return 0
    lax.fori_loop(0, CHUNK, zcrow, 0)

    r0 = sub * ROWS_PT
    pltpu.sync_copy(dst_hbm.at[sub], dst_v)

    def gwait(buf, s):
        pltpu.make_async_copy(xt_hbm.at[pl.ds(0, CHUNK)], buf, s).wait()

    for p in range(2):
        for off, sz in _WB_CHUNKS:
            pltpu.sync_copy(zrows_v.at[pl.ds(0, sz)],
                            s_sh.at[pl.ds(r0 + off, sz)])
        if p == 0:
            for off, sz in _WB_CHUNKS:
                pltpu.sync_copy(zcnt_v.at[pl.ds(0, sz)],
                                cnt_sh.at[pl.ds(r0 + off, sz)])
        plsc.subcore_barrier()

        fi = (core * 2 + p) * NS + sub
        pltpu.sync_copy(src_hbm.at[fi], src_v)

        do_cnt = (p == 0)

        pltpu.async_copy(xt_hbm.at[src_v.at[0]], rows0_v, g0)

        def body2(h, _):
            c = h * 2
            pltpu.async_copy(xt_hbm.at[src_v.at[c + 1]], rows1_v, g1)
            gwait(rows0_v, g0)
            pltpu.sync_copy(rows0_v, s_sh.at[dst_v.at[c]], add=True)
            if do_cnt:
                pltpu.sync_copy(ones_v, cnt_sh.at[dst_v.at[c]], add=True)
            nxt = jnp.minimum(c + 2, NCHUNK - 2)
            pltpu.async_copy(xt_hbm.at[src_v.at[nxt]], rows0_v, g0)
            gwait(rows1_v, g1)
            pltpu.sync_copy(rows1_v, s_sh.at[dst_v.at[c + 1]], add=True)
            if do_cnt:
                pltpu.sync_copy(ones_v, cnt_sh.at[dst_v.at[c + 1]], add=True)
            return 0
        lax.fori_loop(0, NCHUNK // 2, body2, 0)
        gwait(rows0_v, g0)

        plsc.subcore_barrier()

        qbase = (core * 2 + p) * N_PAD + r0
        for off, sz in _WB_CHUNKS:
            pltpu.sync_copy(s_sh.at[pl.ds(r0 + off, sz)],
                            rows0_v.at[pl.ds(0, sz)])
            pltpu.sync_copy(rows0_v.at[pl.ds(0, sz)],
                            s_out.at[pl.ds(qbase + off, sz)])
        if p == 0:
            cbase = core * N_PAD + r0
            for off, sz in _WB_CHUNKS:
                pltpu.sync_copy(cnt_sh.at[pl.ds(r0 + off, sz)],
                                zcnt_v.at[pl.ds(0, sz)])
                pltpu.sync_copy(zcnt_v.at[pl.ds(0, sz)],
                                cnt_out.at[pl.ds(cbase + off, sz)])
        plsc.subcore_barrier()


_sc_accumulate = pl.kernel(
    _sc_body,
    out_type=[
        jax.ShapeDtypeStruct((4 * N_PAD, CQ), jnp.float32),
        jax.ShapeDtypeStruct((NC * N_PAD, 16), jnp.float32),
    ],
    mesh=plsc.VectorSubcoreMesh(
        core_axis_name="c", subcore_axis_name="s",
        num_cores=NC, num_subcores=NS),
    scratch_types=[
        pltpu.VMEM((NCHUNK, CHUNK), jnp.int32),
        pltpu.VMEM((NCHUNK, CHUNK), jnp.int32),
        pltpu.VMEM((CHUNK, CQ), jnp.float32),
        pltpu.VMEM((CHUNK, CQ), jnp.float32),
        pltpu.VMEM((CHUNK, 16), jnp.float32),
        pltpu.VMEM((CHUNK, CQ), jnp.float32),
        pltpu.VMEM((CHUNK, 16), jnp.float32),
        pltpu.VMEM_SHARED((N_PAD, CQ), jnp.float32),
        pltpu.VMEM_SHARED((N_PAD, 16), jnp.float32),
        pltpu.SemaphoreType.DMA,
        pltpu.SemaphoreType.DMA,
    ],
    compiler_params=pltpu.CompilerParams(use_tc_tiling_on_sc=False),
)

_NBLK = 128
_NB = -(-N_PAD // _NBLK)


def _combine_body(x_ref, s_ref, cnt_ref, eye_ref, o_ref):
    ident = eye_ref[...]
    dn = (((0,), (0,)), ((), ()))
    hp = jax.lax.Precision.HIGHEST
    feats = jnp.concatenate(
        [lax.dot_general(s_ref[q], ident, dn, precision=hp) for q in range(4)],
        axis=0)
    cnt_t = lax.dot_general(cnt_ref[:, 0:1], ident, dn, precision=hp)
    o_ref[...] = (cnt_t * x_ref[0] - feats)[None]


_combine = pl.pallas_call(
    _combine_body,
    grid=(_NB,),
    in_specs=[
        pl.BlockSpec((1, C, _NBLK), lambda i: (0, 0, i)),
        pl.BlockSpec((4, _NBLK, CQ), lambda i: (0, i, 0)),
        pl.BlockSpec((_NBLK, 16), lambda i: (i, 0)),
        pl.BlockSpec((_NBLK, _NBLK), lambda i: (0, 0)),
    ],
    out_specs=pl.BlockSpec((1, C, _NBLK), lambda i: (0, 0, i)),
    out_shape=jax.ShapeDtypeStruct((1, C, N), jnp.float32),
)


@jax.jit
def kernel(x, iInd, jInd):
    xt = jnp.zeros((N_PAD, C), jnp.float32).at[:N].set(x[0].T)
    xtc = xt.reshape(N_PAD, 4, CQ).transpose(1, 0, 2).reshape(4 * N_PAD, CQ)

    pad = jnp.full((EP - 2 * E,), N, jnp.int32)
    src0 = jnp.concatenate([jInd, iInd, pad])
    src = (src0[None] + N_PAD * jnp.arange(4, dtype=jnp.int32)[:, None])
    src = src.reshape(4 * NS, NCHUNK, CHUNK)
    dst = jnp.concatenate([iInd, jInd, pad]).reshape(NS, NCHUNK, CHUNK)

    s_p, cnt_p = _sc_accumulate(xtc, src, dst)
    s_q = s_p.reshape(4, N_PAD, CQ)
    eye = jnp.eye(_NBLK, dtype=jnp.float32)
    return _combine(x, s_q, cnt_p, eye)


# speedup vs baseline: 5.7725x; 1.0223x over previous
import math

import numpy as np

import jax
import jax.numpy as jnp
from jax import lax
from jax.experimental import pallas as pl
from jax.experimental.pallas import tpu as pltpu
from jax.experimental.pallas import tpu_sc as plsc

C = 128
CQ = C // 4
N = 10000
E = 320000
NC, NS = 2, 16
CHUNK = 128
NCHUNK = 2 * math.ceil(2 * E / (NS * CHUNK * 2))
EP = NS * CHUNK * NCHUNK
ROWS_PT = 626
N_PAD = NS * ROWS_PT

_WB_CHUNKS = ((0, 128), (128, 128), (256, 128), (384, 128), (512, 114))


def _sc_body(xt_hbm, src_hbm, dst_hbm, s_out, cnt_out,
             src_v, dst_v, ig0_v, ig1_v, rows0_v, rows1_v,
             ones_v, zrows_v, zcnt_v, s_sh, cnt_sh, g0, g1):
    core = lax.axis_index("c")
    sub = lax.axis_index("s")

    zero16 = jnp.zeros((16,), jnp.float32)
    one16 = jnp.ones((16,), jnp.float32)

    def zrow(r, _):
        zrows_v[r, pl.ds(0, 16)] = zero16
        zrows_v[r, pl.ds(16, 16)] = zero16
        return 0
    lax.fori_loop(0, CHUNK, zrow, 0)

    def zcrow(r, _):
        zcnt_v[r, :] = zero16
        ones_v[r, :] = one16
        return 0
    lax.fori_loop(0, CHUNK, zcrow, 0)

    r0 = sub * ROWS_PT
    pltpu.sync_copy(dst_hbm.at[sub], dst_v)
    pltpu.sync_copy(src_hbm.at[sub], src_v)

    def gwait(buf, s):
        pltpu.make_async_copy(xt_hbm.at[pl.ds(0, CHUNK)], buf, s).wait()

    for p in range(2):
        for off, sz in _WB_CHUNKS:
            pltpu.sync_copy(zrows_v.at[pl.ds(0, sz)],
                            s_sh.at[pl.ds(r0 + off, sz)])
        if p == 0:
            for off, sz in _WB_CHUNKS:
                pltpu.sync_copy(zcnt_v.at[pl.ds(0, sz)],
                                cnt_sh.at[pl.ds(r0 + off, sz)])
        plsc.subcore_barrier()

        do_cnt = (p == 0)
        qoff = ((core * 2 + p) * N_PAD).astype(jnp.int32)
        off16 = jnp.zeros((16,), jnp.int32) + qoff

        def fill_ig(ig, c):
            for w in range(CHUNK // 16):
                sl = pl.ds(w * 16, 16)
                ig[sl] = src_v[c, sl] + off16

        fill_ig(ig0_v, 0)
        pltpu.async_copy(xt_hbm.at[ig0_v], rows0_v, g0)

        def body2(h, _):
            c = h * 2
            fill_ig(ig1_v, c + 1)
            pltpu.async_copy(xt_hbm.at[ig1_v], rows1_v, g1)
            gwait(rows0_v, g0)
            pltpu.sync_copy(rows0_v, s_sh.at[dst_v.at[c]], add=True)
            if do_cnt:
                pltpu.sync_copy(ones_v, cnt_sh.at[dst_v.at[c]], add=True)
            nxt = jnp.minimum(c + 2, NCHUNK - 2)
            fill_ig(ig0_v, nxt)
            pltpu.async_copy(xt_hbm.at[ig0_v], rows0_v, g0)
            gwait(rows1_v, g1)
            pltpu.sync_copy(rows1_v, s_sh.at[dst_v.at[c + 1]], add=True)
            if do_cnt:
                pltpu.sync_copy(ones_v, cnt_sh.at[dst_v.at[c + 1]], add=True)
            return 0
        lax.fori_loop(0, NCHUNK // 2, body2, 0)
        gwait(rows0_v, g0)

        plsc.subcore_barrier()

        qbase = (core * 2 + p) * N_PAD + r0
        for off, sz in _WB_CHUNKS:
            pltpu.sync_copy(s_sh.at[pl.ds(r0 + off, sz)],
                            rows0_v.at[pl.ds(0, sz)])
            pltpu.sync_copy(rows0_v.at[pl.ds(0, sz)],
                            s_out.at[pl.ds(qbase + off, sz)])
        if p == 0:
            cbase = core * N_PAD + r0
            for off, sz in _WB_CHUNKS:
                pltpu.sync_copy(cnt_sh.at[pl.ds(r0 + off, sz)],
                                zcnt_v.at[pl.ds(0, sz)])
                pltpu.sync_copy(zcnt_v.at[pl.ds(0, sz)],
                                cnt_out.at[pl.ds(cbase + off, sz)])
        plsc.subcore_barrier()


_sc_accumulate = pl.kernel(
    _sc_body,
    out_type=[
        jax.ShapeDtypeStruct((4 * N_PAD, CQ), jnp.float32),
        jax.ShapeDtypeStruct((NC * N_PAD, 16), jnp.float32),
    ],
    mesh=plsc.VectorSubcoreMesh(
        core_axis_name="c", subcore_axis_name="s",
        num_cores=NC, num_subcores=NS),
    scratch_types=[
        pltpu.VMEM((NCHUNK, CHUNK), jnp.int32),
        pltpu.VMEM((NCHUNK, CHUNK), jnp.int32),
        pltpu.VMEM((CHUNK,), jnp.int32),
        pltpu.VMEM((CHUNK,), jnp.int32),
        pltpu.VMEM((CHUNK, CQ), jnp.float32),
        pltpu.VMEM((CHUNK, CQ), jnp.float32),
        pltpu.VMEM((CHUNK, 16), jnp.float32),
        pltpu.VMEM((CHUNK, CQ), jnp.float32),
        pltpu.VMEM((CHUNK, 16), jnp.float32),
        pltpu.VMEM_SHARED((N_PAD, CQ), jnp.float32),
        pltpu.VMEM_SHARED((N_PAD, 16), jnp.float32),
        pltpu.SemaphoreType.DMA,
        pltpu.SemaphoreType.DMA,
    ],
    compiler_params=pltpu.CompilerParams(use_tc_tiling_on_sc=False),
)

_NBLK = 128
_NB = -(-N_PAD // _NBLK)


def _combine_body(x_ref, s_ref, cnt_ref, eye_ref, o_ref):
    ident = eye_ref[...]
    dn = (((0,), (0,)), ((), ()))
    hp = jax.lax.Precision.HIGHEST
    feats = jnp.concatenate(
        [lax.dot_general(s_ref[q], ident, dn, precision=hp) for q in range(4)],
        axis=0)
    cnt_t = lax.dot_general(cnt_ref[:, 0:1], ident, dn, precision=hp)
    o_ref[...] = (cnt_t * x_ref[0] - feats)[None]


_combine = pl.pallas_call(
    _combine_body,
    grid=(_NB,),
    in_specs=[
        pl.BlockSpec((1, C, _NBLK), lambda i: (0, 0, i)),
        pl.BlockSpec((4, _NBLK, CQ), lambda i: (0, i, 0)),
        pl.BlockSpec((_NBLK, 16), lambda i: (i, 0)),
        pl.BlockSpec((_NBLK, _NBLK), lambda i: (0, 0)),
    ],
    out_specs=pl.BlockSpec((1, C, _NBLK), lambda i: (0, 0, i)),
    out_shape=jax.ShapeDtypeStruct((1, C, N), jnp.float32),
)


_EYE = np.eye(_NBLK, dtype=np.float32)


@jax.jit
def kernel(x, iInd, jInd):
    xp = jnp.pad(x[0], ((0, 0), (0, N_PAD - N)))
    xtc = xp.reshape(4, CQ, N_PAD).transpose(0, 2, 1).reshape(4 * N_PAD, CQ)

    pad = jnp.full((EP - 2 * E,), N, jnp.int32)
    src = jnp.concatenate([jInd, iInd, pad]).reshape(NS, NCHUNK, CHUNK)
    dst = jnp.concatenate([iInd, jInd, pad]).reshape(NS, NCHUNK, CHUNK)

    s_p, cnt_p = _sc_accumulate(xtc, src, dst)
    s_q = s_p.reshape(4, N_PAD, CQ)
    return _combine(x, s_q, cnt_p, _EYE)
